# baseline (speedup 1.0000x reference)
import functools

import jax
import jax.numpy as jnp
from jax import lax
from jax.experimental import pallas as pl
from jax.experimental.pallas import tpu as pltpu
from jax.experimental.pallas import tpu_sc as plsc

D = 128
B = 4096
L = 200
NC = 2
NS = 16
NW = NC * NS
BPW = B // NW
NCH = 4
CH = L // NCH
VL = 16
NV = D // VL

_mesh = plsc.VectorSubcoreMesh(core_axis_name="c", subcore_axis_name="s")


@functools.partial(
    pl.kernel,
    mesh=_mesh,
    out_type=jax.ShapeDtypeStruct((B, D), jnp.float32),
    scratch_types=[
        pltpu.VMEM((BPW, NCH, CH), jnp.int32),
        pltpu.VMEM((L, D), jnp.float32),
        pltpu.VMEM((L, D), jnp.float32),
        pltpu.VMEM((16, D), jnp.float32),
        pltpu.SemaphoreType.DMA,
        pltpu.SemaphoreType.DMA,
    ],
)
def _pooled_lookup(idx_hbm, table_hbm, out_hbm, idx_v, rows0, rows1, out_v,
                   sem0, sem1):
    wid = lax.axis_index("s") * NC + lax.axis_index("c")
    base = wid * BPW

    def start_gather(b, rows, sem):
        for j in range(NCH):
            pltpu.async_copy(
                table_hbm.at[idx_v.at[b, j]],
                rows.at[pl.ds(j * CH, CH)],
                sem,
            )

    def drain_gather(rows, sem):
        pltpu.make_async_copy(table_hbm.at[pl.ds(0, L)], rows, sem).wait()

    def accumulate(rows, b):
        def acc_body(t, acc):
            return tuple(
                acc[j] + rows[t, pl.ds(j * VL, VL)] for j in range(NV)
            )

        acc = lax.fori_loop(
            0, L, acc_body,
            tuple(jnp.zeros((VL,), jnp.float32) for _ in range(NV)),
            unroll=8,
        )
        scale = jnp.float32(1.0 / L)
        for j in range(NV):
            out_v[b % 16, pl.ds(j * VL, VL)] = acc[j] * scale

    pltpu.sync_copy(idx_hbm.at[pl.ds(base, BPW)], idx_v)
    start_gather(0, rows0, sem0)

    def per_pair(g, carry):
        b0 = 2 * g
        b1 = b0 + 1
        start_gather(b1, rows1, sem1)
        drain_gather(rows0, sem0)
        accumulate(rows0, b0)

        @pl.when(b1 + 1 < BPW)
        def _():
            start_gather(b1 + 1, rows0, sem0)

        drain_gather(rows1, sem1)
        accumulate(rows1, b1)

        @pl.when(b1 % 16 == 15)
        def _():
            off = pl.multiple_of(base + b1 - 15, 16)
            pltpu.sync_copy(out_v, out_hbm.at[pl.ds(off, 16)])

        return carry

    lax.fori_loop(0, BPW // 2, per_pair, 0)


def kernel(indices, table):
    idx3 = indices.reshape(B, NCH, CH).astype(jnp.int32)
    return _pooled_lookup(idx3, table)


# speedup vs baseline: 1.0102x; 1.0102x over previous
import functools

import jax
import jax.numpy as jnp
from jax import lax
from jax.experimental import pallas as pl
from jax.experimental.pallas import tpu as pltpu
from jax.experimental.pallas import tpu_sc as plsc

D = 128
B = 4096
L = 200
NC = 2
NS = 16
NW = NC * NS
BPW = B // NW
NCH = 2
CH = L // NCH
VL = 16
NV = D // VL

_mesh = plsc.VectorSubcoreMesh(core_axis_name="c", subcore_axis_name="s")


@functools.partial(
    pl.kernel,
    mesh=_mesh,
    out_type=jax.ShapeDtypeStruct((B, D), jnp.float32),
    scratch_types=[
        pltpu.VMEM((BPW, NCH, CH), jnp.int32),
        pltpu.VMEM((L, D), jnp.float32),
        pltpu.VMEM((L, D), jnp.float32),
        pltpu.VMEM((BPW, D), jnp.float32),
        pltpu.SemaphoreType.DMA,
        pltpu.SemaphoreType.DMA,
    ],
)
def _pooled_lookup(idx_hbm, table_hbm, out_hbm, idx_v, rows0, rows1, out_v,
                   sem0, sem1):
    wid = lax.axis_index("s") * NC + lax.axis_index("c")
    base = wid * BPW

    def start_gather(b, rows, sem):
        for j in range(NCH):
            pltpu.async_copy(
                table_hbm.at[idx_v.at[b, j]],
                rows.at[pl.ds(j * CH, CH)],
                sem,
            )

    def drain_gather(rows, sem):
        pltpu.make_async_copy(table_hbm.at[pl.ds(0, L)], rows, sem).wait()

    def accumulate(rows, b):
        def acc_body(t, acc):
            return tuple(
                acc[j] + rows[t, pl.ds(j * VL, VL)] for j in range(NV)
            )

        acc = lax.fori_loop(
            0, L, acc_body,
            tuple(jnp.zeros((VL,), jnp.float32) for _ in range(NV)),
            unroll=8,
        )
        scale = jnp.float32(1.0 / L)
        for j in range(NV):
            out_v[b, pl.ds(j * VL, VL)] = acc[j] * scale

    pltpu.sync_copy(idx_hbm.at[pl.ds(base, BPW)], idx_v)
    start_gather(0, rows0, sem0)

    def per_pair(g, carry):
        b0 = 2 * g
        b1 = b0 + 1
        start_gather(b1, rows1, sem1)
        drain_gather(rows0, sem0)
        accumulate(rows0, b0)

        @pl.when(b1 + 1 < BPW)
        def _():
            start_gather(b1 + 1, rows0, sem0)

        drain_gather(rows1, sem1)
        accumulate(rows1, b1)
        return carry

    lax.fori_loop(0, BPW // 2, per_pair, 0)
    pltpu.sync_copy(out_v, out_hbm.at[pl.ds(base, BPW)])


def kernel(indices, table):
    idx3 = indices.reshape(B, NCH, CH).astype(jnp.int32)
    return _pooled_lookup(idx3, table)
